# baseline (speedup 1.0000x reference)
import functools

import jax
import jax.numpy as jnp
from jax import lax
from jax.experimental import pallas as pl
from jax.experimental.pallas import tpu as pltpu
from jax.experimental.pallas import tpu_sc as plsc

_L = 16


def _make_sc_kernel(B, S, D):
    info = plsc.get_sparse_core_info()
    NC, NS = info.num_cores, info.num_subcores
    NW = NC * NS
    SW = S // NW
    R = 16
    CH = R * D
    n_chunks = SW // R

    mesh = plsc.VectorSubcoreMesh(core_axis_name="c", subcore_axis_name="s")

    @functools.partial(
        pl.kernel,
        out_type=jax.ShapeDtypeStruct((B * S * D,), jnp.float32),
        mesh=mesh,
        scratch_types=[
            pltpu.VMEM((CH,), jnp.float32),
            pltpu.VMEM((CH,), jnp.float32),
        ],
    )
    def body(x_hbm, pos_hbm, out_hbm, pbuf, xbuf):
        wid = lax.axis_index("s") * NC + lax.axis_index("c")
        base = wid * (SW * D)
        for c in range(n_chunks):
            p_off = base + c * CH
            pltpu.sync_copy(pos_hbm.at[pl.ds(p_off, CH)], pbuf)
            for b in range(B):
                x_off = b * (S * D) + p_off
                pltpu.sync_copy(x_hbm.at[pl.ds(x_off, CH)], xbuf)

                @pl.loop(0, CH // _L)
                def _add(i):
                    sl = pl.ds(i * _L, _L)
                    xbuf[sl] = xbuf[sl] + pbuf[sl]

                pltpu.sync_copy(xbuf, out_hbm.at[pl.ds(x_off, CH)])

    return body


def kernel(x, pos_table):
    B, S, D = x.shape
    sc = _make_sc_kernel(B, S, D)
    out = sc(x.reshape(-1), pos_table.reshape(-1))
    return out.reshape(B, S, D)


# speedup vs baseline: 1.4147x; 1.4147x over previous
import functools

import jax
import jax.numpy as jnp
from jax import lax
from jax.experimental import pallas as pl
from jax.experimental.pallas import tpu as pltpu
from jax.experimental.pallas import tpu_sc as plsc

_L = 16
_NB = 2


def _make_sc_kernel(B, S, D):
    info = plsc.get_sparse_core_info()
    NC, NS = info.num_cores, info.num_subcores
    NW = NC * NS
    SW = S // NW
    R = 16
    CH = R * D
    n_chunks = SW // R
    n_jobs = n_chunks * B

    mesh = plsc.VectorSubcoreMesh(core_axis_name="c", subcore_axis_name="s")

    @functools.partial(
        pl.kernel,
        out_type=jax.ShapeDtypeStruct((B * S * D,), jnp.float32),
        mesh=mesh,
        scratch_types=[
            pltpu.VMEM((2, CH), jnp.float32),
            pltpu.VMEM((_NB, CH), jnp.float32),
            pltpu.SemaphoreType.DMA,
            pltpu.SemaphoreType.DMA,
            pltpu.SemaphoreType.DMA,
        ],
    )
    def body(x_hbm, pos_hbm, out_hbm, pbuf, xbufs, in_sem, out_sem, p_sem):
        wid = lax.axis_index("s") * NC + lax.axis_index("c")
        base = wid * (SW * D)

        def x_off(j):
            c, b = divmod(j, B)
            return b * (S * D) + base + c * CH

        def in_cp(j, k):
            return pltpu.make_async_copy(
                x_hbm.at[pl.ds(x_off(j), CH)], xbufs.at[k], in_sem)

        def out_cp(j, k):
            return pltpu.make_async_copy(
                xbufs.at[k], out_hbm.at[pl.ds(x_off(j), CH)], out_sem)

        def p_cp(c, k):
            return pltpu.make_async_copy(
                pos_hbm.at[pl.ds(base + c * CH, CH)], pbuf.at[k], p_sem)

        p_cp(0, 0).start()
        in_cp(0, 0).start()
        for j in range(n_jobs):
            k = j % _NB
            c, b = divmod(j, B)
            if b == 0:
                p_cp(c, c % 2).wait()
                if c + 1 < n_chunks:
                    p_cp(c + 1, (c + 1) % 2).start()
            in_cp(j, k).wait()

            xb = xbufs.at[k]
            pb = pbuf.at[c % 2]

            @plsc.parallel_loop(0, CH // _L, unroll=8)
            def _add(i):
                sl = pl.ds(i * _L, _L)
                xb[sl] = xb[sl] + pb[sl]

            out_cp(j, k).start()
            if j + 1 < n_jobs:
                if j + 1 - _NB >= 0:
                    out_cp(j + 1 - _NB, (j + 1) % _NB).wait()
                in_cp(j + 1, (j + 1) % _NB).start()
        for j in range(max(0, n_jobs - _NB), n_jobs):
            out_cp(j, j % _NB).wait()

    return body


def kernel(x, pos_table):
    B, S, D = x.shape
    sc = _make_sc_kernel(B, S, D)
    out = sc(x.reshape(-1), pos_table.reshape(-1))
    return out.reshape(B, S, D)


# speedup vs baseline: 3.3670x; 2.3799x over previous
import functools

import jax
import jax.numpy as jnp
from jax import lax
from jax.experimental import pallas as pl
from jax.experimental.pallas import tpu as pltpu
from jax.experimental.pallas import tpu_sc as plsc

_L = 16
_NB = 2


def _make_sc_kernel(B, S, D):
    info = plsc.get_sparse_core_info()
    NC, NS = info.num_cores, info.num_subcores
    NW = NC * NS
    SW = S // NW
    R = 16
    n_chunks = SW // R
    n_jobs = n_chunks * B

    mesh = plsc.VectorSubcoreMesh(core_axis_name="c", subcore_axis_name="s")

    @functools.partial(
        pl.kernel,
        out_type=jax.ShapeDtypeStruct((B, S, D), jnp.float32),
        mesh=mesh,
        scratch_types=[
            pltpu.VMEM((2, R, D), jnp.float32),
            pltpu.VMEM((_NB, R, D), jnp.float32),
            pltpu.SemaphoreType.DMA,
            pltpu.SemaphoreType.DMA,
            pltpu.SemaphoreType.DMA,
        ],
        compiler_params=pltpu.CompilerParams(use_tc_tiling_on_sc=True),
    )
    def body(x_hbm, pos_hbm, out_hbm, pbuf, xbufs, in_sem, out_sem, p_sem):
        wid = lax.axis_index("s") * NC + lax.axis_index("c")
        base = wid * SW

        def rows(j):
            c, b = divmod(j, B)
            return b, base + c * R

        def in_cp(j, k):
            b, r0 = rows(j)
            return pltpu.make_async_copy(
                x_hbm.at[b, pl.ds(r0, R), :], xbufs.at[k], in_sem)

        def out_cp(j, k):
            b, r0 = rows(j)
            return pltpu.make_async_copy(
                xbufs.at[k], out_hbm.at[b, pl.ds(r0, R), :], out_sem)

        def p_cp(c, k):
            return pltpu.make_async_copy(
                pos_hbm.at[pl.ds(base + c * R, R), :], pbuf.at[k], p_sem)

        p_cp(0, 0).start()
        in_cp(0, 0).start()
        for j in range(n_jobs):
            k = j % _NB
            c, b = divmod(j, B)
            if b == 0:
                p_cp(c, c % 2).wait()
                if c + 1 < n_chunks:
                    p_cp(c + 1, (c + 1) % 2).start()
            in_cp(j, k).wait()

            xb = xbufs.at[k]
            pb = pbuf.at[c % 2]

            @plsc.parallel_loop(0, (R * D) // _L, unroll=8)
            def _add(i):
                r = i >> 6
                sl = pl.ds((i & (D // _L - 1)) * _L, _L)
                xb[r, sl] = xb[r, sl] + pb[r, sl]

            out_cp(j, k).start()
            if j + 1 < n_jobs:
                if j + 1 - _NB >= 0:
                    out_cp(j + 1 - _NB, (j + 1) % _NB).wait()
                in_cp(j + 1, (j + 1) % _NB).start()
        for j in range(max(0, n_jobs - _NB), n_jobs):
            out_cp(j, j % _NB).wait()

    return body


def kernel(x, pos_table):
    B, S, D = x.shape
    sc = _make_sc_kernel(B, S, D)
    return sc(x, pos_table)


# speedup vs baseline: 3.6987x; 1.0985x over previous
import functools

import jax
import jax.numpy as jnp
from jax import lax
from jax.experimental import pallas as pl
from jax.experimental.pallas import tpu as pltpu
from jax.experimental.pallas import tpu_sc as plsc

_L = 16
_NB = 2


def _make_sc_kernel(B, S, D):
    info = plsc.get_sparse_core_info()
    NC, NS = info.num_cores, info.num_subcores
    NW = NC * NS
    SW = S // NW
    R = 32
    n_chunks = SW // R
    n_jobs = n_chunks * B

    mesh = plsc.VectorSubcoreMesh(core_axis_name="c", subcore_axis_name="s")

    @functools.partial(
        pl.kernel,
        out_type=jax.ShapeDtypeStruct((B, S, D), jnp.float32),
        mesh=mesh,
        scratch_types=[
            pltpu.VMEM((1, R, D), jnp.float32),
            pltpu.VMEM((_NB, R, D), jnp.float32),
            pltpu.SemaphoreType.DMA,
            pltpu.SemaphoreType.DMA,
            pltpu.SemaphoreType.DMA,
        ],
        compiler_params=pltpu.CompilerParams(use_tc_tiling_on_sc=True),
    )
    def body(x_hbm, pos_hbm, out_hbm, pbuf, xbufs, in_sem, out_sem, p_sem):
        wid = lax.axis_index("s") * NC + lax.axis_index("c")
        base = wid * SW

        def rows(j):
            c, b = divmod(j, B)
            return b, base + c * R

        def in_cp(j, k):
            b, r0 = rows(j)
            return pltpu.make_async_copy(
                x_hbm.at[b, pl.ds(r0, R), :], xbufs.at[k], in_sem)

        def out_cp(j, k):
            b, r0 = rows(j)
            return pltpu.make_async_copy(
                xbufs.at[k], out_hbm.at[b, pl.ds(r0, R), :], out_sem)

        def p_cp(c, k):
            return pltpu.make_async_copy(
                pos_hbm.at[pl.ds(base + c * R, R), :], pbuf.at[k], p_sem)

        p_cp(0, 0).start()
        in_cp(0, 0).start()
        for j in range(n_jobs):
            k = j % _NB
            c, b = divmod(j, B)
            if b == 0:
                p_cp(c, 0).wait()
            in_cp(j, k).wait()

            xb = xbufs.at[k]
            pb = pbuf.at[0]

            @plsc.parallel_loop(0, (R * D) // _L, unroll=8)
            def _add(i):
                r = i >> 6
                sl = pl.ds((i & (D // _L - 1)) * _L, _L)
                xb[r, sl] = xb[r, sl] + pb[r, sl]

            if b == B - 1 and c + 1 < n_chunks:
                p_cp(c + 1, 0).start()
            out_cp(j, k).start()
            if j + 1 < n_jobs:
                if j + 1 - _NB >= 0:
                    out_cp(j + 1 - _NB, (j + 1) % _NB).wait()
                in_cp(j + 1, (j + 1) % _NB).start()
        for j in range(max(0, n_jobs - _NB), n_jobs):
            out_cp(j, j % _NB).wait()

    return body


def kernel(x, pos_table):
    B, S, D = x.shape
    sc = _make_sc_kernel(B, S, D)
    return sc(x, pos_table)


# speedup vs baseline: 4.0285x; 1.0892x over previous
import functools

import jax
import jax.numpy as jnp
from jax import lax
from jax.experimental import pallas as pl
from jax.experimental.pallas import tpu as pltpu
from jax.experimental.pallas import tpu_sc as plsc

_L = 16
_NB = 2


def _make_sc_kernel(B, S, D):
    info = plsc.get_sparse_core_info()
    NC, NS = info.num_cores, info.num_subcores
    NW = NC * NS
    SW = S // NW
    R = 8
    n_chunks = SW // R

    mesh = plsc.VectorSubcoreMesh(core_axis_name="c", subcore_axis_name="s")

    @functools.partial(
        pl.kernel,
        out_type=jax.ShapeDtypeStruct((B, S, D), jnp.float32),
        mesh=mesh,
        scratch_types=[
            pltpu.VMEM((2, R, D), jnp.float32),
            pltpu.VMEM((_NB, B, R, D), jnp.float32),
            pltpu.SemaphoreType.DMA,
            pltpu.SemaphoreType.DMA,
            pltpu.SemaphoreType.DMA,
        ],
        compiler_params=pltpu.CompilerParams(use_tc_tiling_on_sc=True),
    )
    def body(x_hbm, pos_hbm, out_hbm, pbuf, xbufs, in_sem, out_sem, p_sem):
        wid = lax.axis_index("s") * NC + lax.axis_index("c")
        base = wid * SW

        def in_cp(c, k):
            return pltpu.make_async_copy(
                x_hbm.at[:, pl.ds(base + c * R, R), :], xbufs.at[k], in_sem)

        def out_cp(c, k):
            return pltpu.make_async_copy(
                xbufs.at[k], out_hbm.at[:, pl.ds(base + c * R, R), :], out_sem)

        def p_cp(c, k):
            return pltpu.make_async_copy(
                pos_hbm.at[pl.ds(base + c * R, R), :], pbuf.at[k], p_sem)

        p_cp(0, 0).start()
        in_cp(0, 0).start()
        for c in range(n_chunks):
            k = c % _NB
            p_cp(c, c % 2).wait()
            if c + 1 < n_chunks:
                p_cp(c + 1, (c + 1) % 2).start()
            in_cp(c, k).wait()

            xb = xbufs.at[k]
            pb = pbuf.at[c % 2]

            @plsc.parallel_loop(0, (R * D) // _L, unroll=4)
            def _add(i):
                r = i >> 6
                sl = pl.ds((i & (D // _L - 1)) * _L, _L)
                pv = pb[r, sl]
                for b in range(B):
                    xb[b, r, sl] = xb[b, r, sl] + pv

            out_cp(c, k).start()
            if c + 1 < n_chunks:
                if c + 1 - _NB >= 0:
                    out_cp(c + 1 - _NB, (c + 1) % _NB).wait()
                in_cp(c + 1, (c + 1) % _NB).start()
        for c in range(max(0, n_chunks - _NB), n_chunks):
            out_cp(c, c % _NB).wait()

    return body


def kernel(x, pos_table):
    B, S, D = x.shape
    sc = _make_sc_kernel(B, S, D)
    return sc(x, pos_table)
